# baseline (device time: 899708 ns/iter reference)
import jax
import jax.numpy as jnp
from jax import lax
from jax.experimental import pallas as pl
from jax.experimental.pallas import tpu as pltpu

N_DEV = 32


def kernel(x, Win0, Wout0, Win1, Wout1, Win2, Wout2):
    B, D = x.shape
    H = Win0.shape[1]
    M = N_DEV * B

    def body(x_ref, win0, wout0, win1, wout1, win2, wout2, out_ref,
             xfull, ybuf, rsbuf, ag_send, ag_recv, rs_send, rs_recv):
        i = lax.axis_index("i")
        left = lax.rem(i + N_DEV - 1, N_DEV)
        right = lax.rem(i + 1, N_DEV)

        barrier = pltpu.get_barrier_semaphore()
        for nbr in (left, right):
            pl.semaphore_signal(barrier, inc=1, device_id=(nbr,),
                                device_id_type=pl.DeviceIdType.MESH)
        pl.semaphore_wait(barrier, 2)

        def row_chunk(ref, c):
            return ref.at[pl.ds(c * B, B), :]

        x_bs = x_ref[...]
        for win, wout in ((win0, wout0), (win1, wout1), (win2, wout2)):
            xfull[pl.ds(i * B, B), :] = x_bs
            for h in range(N_DEV - 1):
                send_c = lax.rem(i + 2 * N_DEV - h, N_DEV)
                recv_c = lax.rem(i + 2 * N_DEV - 1 - h, N_DEV)
                send = pltpu.make_async_remote_copy(
                    src_ref=row_chunk(xfull, send_c),
                    dst_ref=row_chunk(xfull, send_c),
                    send_sem=ag_send.at[h % 2],
                    recv_sem=ag_recv.at[send_c],
                    device_id=(right,),
                    device_id_type=pl.DeviceIdType.MESH,
                )
                send.start()
                send.wait_send()
                recv = pltpu.make_async_remote_copy(
                    src_ref=row_chunk(xfull, recv_c),
                    dst_ref=row_chunk(xfull, recv_c),
                    send_sem=ag_send.at[h % 2],
                    recv_sem=ag_recv.at[recv_c],
                    device_id=(left,),
                    device_id_type=pl.DeviceIdType.MESH,
                )
                recv.wait_recv()

            for blk in range(8):
                rows = pl.ds(blk * (M // 8), M // 8)
                hv = jnp.maximum(
                    jnp.dot(xfull[rows, :], win[...],
                            preferred_element_type=jnp.float32), 0.0)
                ybuf[rows, :] = jnp.dot(
                    hv, wout[...], preferred_element_type=jnp.float32)

            for s in range(N_DEV - 1):
                send_c = lax.rem(i + 2 * N_DEV - 1 - s, N_DEV)
                if s == 0:
                    src = row_chunk(ybuf, send_c)
                else:
                    rsbuf[s - 1, :, :] = (
                        rsbuf[s - 1, :, :] + ybuf[pl.ds(send_c * B, B), :])
                    src = rsbuf.at[s - 1]
                send = pltpu.make_async_remote_copy(
                    src_ref=src,
                    dst_ref=rsbuf.at[s],
                    send_sem=rs_send.at[s % 2],
                    recv_sem=rs_recv.at[s],
                    device_id=(right,),
                    device_id_type=pl.DeviceIdType.MESH,
                )
                send.start()
                send.wait_send()
                recv = pltpu.make_async_remote_copy(
                    src_ref=rsbuf.at[s],
                    dst_ref=rsbuf.at[s],
                    send_sem=rs_send.at[s % 2],
                    recv_sem=rs_recv.at[s],
                    device_id=(left,),
                    device_id_type=pl.DeviceIdType.MESH,
                )
                recv.wait_recv()

            x_bs = rsbuf[N_DEV - 2, :, :] + ybuf[pl.ds(i * B, B), :]

        out_ref[...] = x_bs

    return pl.pallas_call(
        body,
        out_shape=jax.ShapeDtypeStruct((B, D), jnp.float32),
        in_specs=[pl.BlockSpec(memory_space=pltpu.VMEM)] * 7,
        out_specs=pl.BlockSpec(memory_space=pltpu.VMEM),
        scratch_shapes=[
            pltpu.VMEM((M, D), jnp.float32),
            pltpu.VMEM((M, D), jnp.float32),
            pltpu.VMEM((N_DEV - 1, B, D), jnp.float32),
            pltpu.SemaphoreType.DMA((2,)),
            pltpu.SemaphoreType.DMA((N_DEV,)),
            pltpu.SemaphoreType.DMA((2,)),
            pltpu.SemaphoreType.DMA((N_DEV - 1,)),
        ],
        compiler_params=pltpu.CompilerParams(collective_id=0),
    )(x, Win0, Wout0, Win1, Wout1, Win2, Wout2)


# device time: 570626 ns/iter; 1.5767x vs baseline; 1.5767x over previous
import jax
import jax.numpy as jnp
from jax import lax
from jax.experimental import pallas as pl
from jax.experimental.pallas import tpu as pltpu

N_DEV = 32
R_HOPS = 16
L_HOPS = 15


def kernel(x, Win0, Wout0, Win1, Wout1, Win2, Wout2):
    B, D = x.shape
    M = N_DEV * B

    def body(x_ref, win0, wout0, win1, wout1, win2, wout2, out_ref,
             xfull, ybuf, rs_r, rs_l,
             ag_sr, ag_sl, rs_sr, rs_sl, ag_recv, rs_rr, rs_rl):
        i = lax.axis_index("i")
        left = lax.rem(i + N_DEV - 1, N_DEV)
        right = lax.rem(i + 1, N_DEV)

        barrier = pltpu.get_barrier_semaphore()
        for nbr in (left, right):
            pl.semaphore_signal(barrier, inc=1, device_id=(nbr,),
                                device_id_type=pl.DeviceIdType.MESH)
        pl.semaphore_wait(barrier, 2)

        def mod(e):
            return lax.rem(e + 2 * N_DEV, N_DEV)

        def chunk(ref, c):
            return ref.at[pl.ds(c * B, B), :]

        def send_to(src, dst, ssem, rsem, dev):
            r = pltpu.make_async_remote_copy(
                src_ref=src, dst_ref=dst, send_sem=ssem, recv_sem=rsem,
                device_id=(dev,), device_id_type=pl.DeviceIdType.MESH)
            r.start()
            return r

        def wait_recv(dst, rsem, dev):
            pltpu.make_async_remote_copy(
                src_ref=dst, dst_ref=dst, send_sem=rsem, recv_sem=rsem,
                device_id=(dev,), device_id_type=pl.DeviceIdType.MESH
            ).wait_recv()

        def drain(sends):
            for s in sends[-2:]:
                s.wait_send()

        x_bs = x_ref[...]
        for win, wout in ((win0, wout0), (win1, wout1), (win2, wout2)):

            def compute_chunk(c, win=win, wout=wout):
                xv = xfull[pl.ds(c * B, B), :]
                hv = jnp.maximum(
                    jnp.dot(xv, win[...],
                            preferred_element_type=jnp.float32), 0.0)
                ybuf[pl.ds(c * B, B), :] = jnp.dot(
                    hv, wout[...], preferred_element_type=jnp.float32)

            xfull[pl.ds(i * B, B), :] = x_bs
            sr = [send_to(chunk(xfull, i), chunk(xfull, i),
                          ag_sr.at[0], ag_recv.at[i], right)]
            sl = [send_to(chunk(xfull, i), chunk(xfull, i),
                          ag_sl.at[0], ag_recv.at[i], left)]
            compute_chunk(i)
            for h in range(R_HOPS):
                c_l = mod(i - 1 - h)
                c_r = mod(i + 1 + h)
                wait_recv(chunk(xfull, c_l), ag_recv.at[c_l], left)
                if h + 1 < R_HOPS:
                    if len(sr) >= 2:
                        sr[-2].wait_send()
                    sr.append(send_to(chunk(xfull, c_l), chunk(xfull, c_l),
                                      ag_sr.at[len(sr) % 2],
                                      ag_recv.at[c_l], right))
                if h < L_HOPS:
                    wait_recv(chunk(xfull, c_r), ag_recv.at[c_r], right)
                if h + 1 < L_HOPS:
                    if len(sl) >= 2:
                        sl[-2].wait_send()
                    sl.append(send_to(chunk(xfull, c_r), chunk(xfull, c_r),
                                      ag_sl.at[len(sl) % 2],
                                      ag_recv.at[c_r], left))
                compute_chunk(c_l)
                if h < L_HOPS:
                    compute_chunk(c_r)
            drain(sr)
            drain(sl)

            rsr = [send_to(chunk(ybuf, mod(i + R_HOPS)), rs_r.at[0],
                           rs_sr.at[0], rs_rr.at[0], right)]
            rsl = [send_to(chunk(ybuf, mod(i - L_HOPS)), rs_l.at[0],
                           rs_sl.at[0], rs_rl.at[0], left)]
            for s in range(R_HOPS):
                wait_recv(rs_r.at[s], rs_rr.at[s], left)
                if s + 1 < R_HOPS:
                    c = mod(i + R_HOPS - 1 - s)
                    rs_r[s, :, :] = rs_r[s, :, :] + ybuf[pl.ds(c * B, B), :]
                    if len(rsr) >= 2:
                        rsr[-2].wait_send()
                    rsr.append(send_to(rs_r.at[s], rs_r.at[s + 1],
                                       rs_sr.at[len(rsr) % 2],
                                       rs_rr.at[s + 1], right))
                if s < L_HOPS:
                    wait_recv(rs_l.at[s], rs_rl.at[s], right)
                if s + 1 < L_HOPS:
                    c = mod(i - L_HOPS + 1 + s)
                    rs_l[s, :, :] = rs_l[s, :, :] + ybuf[pl.ds(c * B, B), :]
                    if len(rsl) >= 2:
                        rsl[-2].wait_send()
                    rsl.append(send_to(rs_l.at[s], rs_l.at[s + 1],
                                       rs_sl.at[len(rsl) % 2],
                                       rs_rl.at[s + 1], left))
            drain(rsr)
            drain(rsl)
            x_bs = (rs_r[R_HOPS - 1, :, :] + rs_l[L_HOPS - 1, :, :]
                    + ybuf[pl.ds(i * B, B), :])

        out_ref[...] = x_bs

    return pl.pallas_call(
        body,
        out_shape=jax.ShapeDtypeStruct((B, D), jnp.float32),
        in_specs=[pl.BlockSpec(memory_space=pltpu.VMEM)] * 7,
        out_specs=pl.BlockSpec(memory_space=pltpu.VMEM),
        scratch_shapes=[
            pltpu.VMEM((M, D), jnp.float32),
            pltpu.VMEM((M, D), jnp.float32),
            pltpu.VMEM((R_HOPS, B, D), jnp.float32),
            pltpu.VMEM((L_HOPS, B, D), jnp.float32),
            pltpu.SemaphoreType.DMA((2,)),
            pltpu.SemaphoreType.DMA((2,)),
            pltpu.SemaphoreType.DMA((2,)),
            pltpu.SemaphoreType.DMA((2,)),
            pltpu.SemaphoreType.DMA((N_DEV,)),
            pltpu.SemaphoreType.DMA((R_HOPS,)),
            pltpu.SemaphoreType.DMA((L_HOPS,)),
        ],
        compiler_params=pltpu.CompilerParams(collective_id=0),
    )(x, Win0, Wout0, Win1, Wout1, Win2, Wout2)


# device time: 376236 ns/iter; 2.3913x vs baseline; 1.5167x over previous
import jax
import jax.numpy as jnp
from jax import lax
from jax.experimental import pallas as pl
from jax.experimental.pallas import tpu as pltpu

N_DEV = 32
R_HOPS = 16
L_HOPS = 15


def kernel(x, Win0, Wout0, Win1, Wout1, Win2, Wout2):
    B, D = x.shape
    M = N_DEV * B

    def body(x_ref, win0, wout0, win1, wout1, win2, wout2, out_ref,
             xfull, ybuf, rs_r, rs_l, stage_r, stage_l,
             ag_sr, ag_sl, rs_sr, rs_sl, ag_recv, rs_rr, rs_rl):
        i = lax.axis_index("i")
        left = lax.rem(i + N_DEV - 1, N_DEV)
        right = lax.rem(i + 1, N_DEV)

        barrier = pltpu.get_barrier_semaphore()
        for nbr in (left, right):
            pl.semaphore_signal(barrier, inc=1, device_id=(nbr,),
                                device_id_type=pl.DeviceIdType.MESH)
        pl.semaphore_wait(barrier, 2)

        def mod(e):
            return lax.rem(e + 2 * N_DEV, N_DEV)

        def chunk(ref, c):
            return ref.at[pl.ds(c * B, B), :]

        def send_to(src, dst, ssem, rsem, dev):
            r = pltpu.make_async_remote_copy(
                src_ref=src, dst_ref=dst, send_sem=ssem, recv_sem=rsem,
                device_id=(dev,), device_id_type=pl.DeviceIdType.MESH)
            r.start()
            return r

        def wait_recv(dst, rsem, dev):
            pltpu.make_async_remote_copy(
                src_ref=dst, dst_ref=dst, send_sem=rsem, recv_sem=rsem,
                device_id=(dev,), device_id_type=pl.DeviceIdType.MESH
            ).wait_recv()

        def drain(sends):
            for s in sends[-2:]:
                s.wait_send()

        x_bs = x_ref[...]
        for win, wout in ((win0, wout0), (win1, wout1), (win2, wout2)):

            def compute_chunk(c, win=win, wout=wout):
                xv = xfull[pl.ds(c * B, B), :].astype(jnp.float32)
                hv = jnp.maximum(
                    jnp.dot(xv, win[...],
                            preferred_element_type=jnp.float32), 0.0)
                ybuf[pl.ds(c * B, B), :] = jnp.dot(
                    hv, wout[...], preferred_element_type=jnp.float32)

            xfull[pl.ds(i * B, B), :] = x_bs.astype(jnp.bfloat16)
            sr = [send_to(chunk(xfull, i), chunk(xfull, i),
                          ag_sr.at[0], ag_recv.at[i], right)]
            sl = [send_to(chunk(xfull, i), chunk(xfull, i),
                          ag_sl.at[0], ag_recv.at[i], left)]
            compute_chunk(i)
            for h in range(R_HOPS):
                c_l = mod(i - 1 - h)
                c_r = mod(i + 1 + h)
                wait_recv(chunk(xfull, c_l), ag_recv.at[c_l], left)
                if h + 1 < R_HOPS:
                    if len(sr) >= 2:
                        sr[-2].wait_send()
                    sr.append(send_to(chunk(xfull, c_l), chunk(xfull, c_l),
                                      ag_sr.at[len(sr) % 2],
                                      ag_recv.at[c_l], right))
                if h < L_HOPS:
                    wait_recv(chunk(xfull, c_r), ag_recv.at[c_r], right)
                if h + 1 < L_HOPS:
                    if len(sl) >= 2:
                        sl[-2].wait_send()
                    sl.append(send_to(chunk(xfull, c_r), chunk(xfull, c_r),
                                      ag_sl.at[len(sl) % 2],
                                      ag_recv.at[c_r], left))
                compute_chunk(c_l)
                if h < L_HOPS:
                    compute_chunk(c_r)
            drain(sr)
            drain(sl)

            stage_r[...] = ybuf[
                pl.ds(mod(i + R_HOPS) * B, B), :].astype(jnp.bfloat16)
            stage_l[...] = ybuf[
                pl.ds(mod(i - L_HOPS) * B, B), :].astype(jnp.bfloat16)
            rsr = [send_to(stage_r, rs_r.at[0],
                           rs_sr.at[0], rs_rr.at[0], right)]
            rsl = [send_to(stage_l, rs_l.at[0],
                           rs_sl.at[0], rs_rl.at[0], left)]
            for s in range(R_HOPS):
                wait_recv(rs_r.at[s], rs_rr.at[s], left)
                if s + 1 < R_HOPS:
                    c = mod(i + R_HOPS - 1 - s)
                    rs_r[s, :, :] = (
                        rs_r[s, :, :].astype(jnp.float32)
                        + ybuf[pl.ds(c * B, B), :]).astype(jnp.bfloat16)
                    if len(rsr) >= 2:
                        rsr[-2].wait_send()
                    rsr.append(send_to(rs_r.at[s], rs_r.at[s + 1],
                                       rs_sr.at[len(rsr) % 2],
                                       rs_rr.at[s + 1], right))
                if s < L_HOPS:
                    wait_recv(rs_l.at[s], rs_rl.at[s], right)
                if s + 1 < L_HOPS:
                    c = mod(i - L_HOPS + 1 + s)
                    rs_l[s, :, :] = (
                        rs_l[s, :, :].astype(jnp.float32)
                        + ybuf[pl.ds(c * B, B), :]).astype(jnp.bfloat16)
                    if len(rsl) >= 2:
                        rsl[-2].wait_send()
                    rsl.append(send_to(rs_l.at[s], rs_l.at[s + 1],
                                       rs_sl.at[len(rsl) % 2],
                                       rs_rl.at[s + 1], left))
            drain(rsr)
            drain(rsl)
            x_bs = (rs_r[R_HOPS - 1, :, :].astype(jnp.float32)
                    + rs_l[L_HOPS - 1, :, :].astype(jnp.float32)
                    + ybuf[pl.ds(i * B, B), :])

        out_ref[...] = x_bs

    return pl.pallas_call(
        body,
        out_shape=jax.ShapeDtypeStruct((B, D), jnp.float32),
        in_specs=[pl.BlockSpec(memory_space=pltpu.VMEM)] * 7,
        out_specs=pl.BlockSpec(memory_space=pltpu.VMEM),
        scratch_shapes=[
            pltpu.VMEM((M, D), jnp.bfloat16),
            pltpu.VMEM((M, D), jnp.float32),
            pltpu.VMEM((R_HOPS, B, D), jnp.bfloat16),
            pltpu.VMEM((L_HOPS, B, D), jnp.bfloat16),
            pltpu.VMEM((B, D), jnp.bfloat16),
            pltpu.VMEM((B, D), jnp.bfloat16),
            pltpu.SemaphoreType.DMA((2,)),
            pltpu.SemaphoreType.DMA((2,)),
            pltpu.SemaphoreType.DMA((2,)),
            pltpu.SemaphoreType.DMA((2,)),
            pltpu.SemaphoreType.DMA((N_DEV,)),
            pltpu.SemaphoreType.DMA((R_HOPS,)),
            pltpu.SemaphoreType.DMA((L_HOPS,)),
        ],
        compiler_params=pltpu.CompilerParams(collective_id=0),
    )(x, Win0, Wout0, Win1, Wout1, Win2, Wout2)


# device time: 311802 ns/iter; 2.8855x vs baseline; 1.2067x over previous
import jax
import jax.numpy as jnp
from jax import lax
from jax.experimental import pallas as pl
from jax.experimental.pallas import tpu as pltpu

N_DEV = 32
R_HOPS = 16
L_HOPS = 15


def _r2i_py(r):
    r %= N_DEV
    q, zz = r // 4, r % 4
    z = zz if q % 2 == 0 else 3 - zz
    x = 0 if q < 4 else 1
    y = q if q < 4 else 7 - q
    xx = x if y % 2 == 0 else 1 - x
    return z * 8 + y * 2 + xx


def _i2r_py(i):
    z, y, xx = i // 8, (i % 8) // 2, i % 2
    x = xx if y % 2 == 0 else 1 - xx
    q = y if x == 0 else 7 - y
    return q * 4 + (z if q % 2 == 0 else 3 - z)


def kernel(x, Win0, Wout0, Win1, Wout1, Win2, Wout2):
    B, D = x.shape
    M = N_DEV * B

    r2i_tab = jnp.array([_r2i_py(r) for r in range(2 * N_DEV)],
                        dtype=jnp.int32)
    i2r_tab = jnp.array([_i2r_py(v) for v in range(N_DEV)],
                        dtype=jnp.int32)
    ii = lax.axis_index("i")
    r = i2r_tab[ii]

    def ring(e):
        return r2i_tab[(e + 2 * N_DEV) % (2 * N_DEV)]

    tab = jnp.concatenate([
        jnp.stack([ring(r - 1),
                   ring(r + 1),
                   ring(r + R_HOPS),
                   ring(r - L_HOPS)]),
        ring(r - 1 - jnp.arange(R_HOPS)),
        ring(r + 1 + jnp.arange(L_HOPS)),
        ring(r + R_HOPS - 1 - jnp.arange(L_HOPS)),
        ring(r - L_HOPS + 1 + jnp.arange(L_HOPS - 1)),
    ]).astype(jnp.int32)

    def body(tab_ref, x_ref, win0, wout0, win1, wout1, win2, wout2,
             out_ref, xfull, ybuf, rs_r, rs_l, stage_r, stage_l,
             ag_sr, ag_sl, rs_sr, rs_sl, ag_recv, rs_rr, rs_rl):
        i = lax.axis_index("i")
        left = tab_ref[0]
        right = tab_ref[1]

        barrier = pltpu.get_barrier_semaphore()
        for nbr in (left, right):
            pl.semaphore_signal(barrier, inc=1, device_id=(nbr,),
                                device_id_type=pl.DeviceIdType.MESH)
        pl.semaphore_wait(barrier, 2)

        def mod(e):
            return lax.rem(e + 2 * N_DEV, N_DEV)

        def chunk(ref, c):
            return ref.at[pl.ds(c * B, B), :]

        def send_to(src, dst, ssem, rsem, dev):
            r = pltpu.make_async_remote_copy(
                src_ref=src, dst_ref=dst, send_sem=ssem, recv_sem=rsem,
                device_id=(dev,), device_id_type=pl.DeviceIdType.MESH)
            r.start()
            return r

        def wait_recv(dst, rsem, dev):
            pltpu.make_async_remote_copy(
                src_ref=dst, dst_ref=dst, send_sem=rsem, recv_sem=rsem,
                device_id=(dev,), device_id_type=pl.DeviceIdType.MESH
            ).wait_recv()

        def drain(sends):
            for s in sends[-2:]:
                s.wait_send()

        x_bs = x_ref[...]
        for win, wout in ((win0, wout0), (win1, wout1), (win2, wout2)):

            def compute_chunk(c, win=win, wout=wout):
                xv = xfull[pl.ds(c * B, B), :].astype(jnp.float32)
                hv = jnp.maximum(
                    jnp.dot(xv, win[...],
                            preferred_element_type=jnp.float32), 0.0)
                ybuf[pl.ds(c * B, B), :] = jnp.dot(
                    hv, wout[...], preferred_element_type=jnp.float32)

            xfull[pl.ds(i * B, B), :] = x_bs.astype(jnp.bfloat16)
            sr = [send_to(chunk(xfull, i), chunk(xfull, i),
                          ag_sr.at[0], ag_recv.at[i], right)]
            sl = [send_to(chunk(xfull, i), chunk(xfull, i),
                          ag_sl.at[0], ag_recv.at[i], left)]
            compute_chunk(i)
            for h in range(R_HOPS):
                c_l = tab_ref[4 + h]
                c_r = tab_ref[20 + h]
                wait_recv(chunk(xfull, c_l), ag_recv.at[c_l], left)
                if h + 1 < R_HOPS:
                    if len(sr) >= 2:
                        sr[-2].wait_send()
                    sr.append(send_to(chunk(xfull, c_l), chunk(xfull, c_l),
                                      ag_sr.at[len(sr) % 2],
                                      ag_recv.at[c_l], right))
                if h < L_HOPS:
                    wait_recv(chunk(xfull, c_r), ag_recv.at[c_r], right)
                if h + 1 < L_HOPS:
                    if len(sl) >= 2:
                        sl[-2].wait_send()
                    sl.append(send_to(chunk(xfull, c_r), chunk(xfull, c_r),
                                      ag_sl.at[len(sl) % 2],
                                      ag_recv.at[c_r], left))
                compute_chunk(c_l)
                if h < L_HOPS:
                    compute_chunk(c_r)
            drain(sr)
            drain(sl)

            stage_r[...] = ybuf[
                pl.ds(tab_ref[2] * B, B), :].astype(jnp.bfloat16)
            stage_l[...] = ybuf[
                pl.ds(tab_ref[3] * B, B), :].astype(jnp.bfloat16)
            rsr = [send_to(stage_r, rs_r.at[0],
                           rs_sr.at[0], rs_rr.at[0], right)]
            rsl = [send_to(stage_l, rs_l.at[0],
                           rs_sl.at[0], rs_rl.at[0], left)]
            for s in range(R_HOPS):
                wait_recv(rs_r.at[s], rs_rr.at[s], left)
                if s + 1 < R_HOPS:
                    c = tab_ref[35 + s]
                    rs_r[s, :, :] = (
                        rs_r[s, :, :].astype(jnp.float32)
                        + ybuf[pl.ds(c * B, B), :]).astype(jnp.bfloat16)
                    if len(rsr) >= 2:
                        rsr[-2].wait_send()
                    rsr.append(send_to(rs_r.at[s], rs_r.at[s + 1],
                                       rs_sr.at[len(rsr) % 2],
                                       rs_rr.at[s + 1], right))
                if s < L_HOPS:
                    wait_recv(rs_l.at[s], rs_rl.at[s], right)
                if s + 1 < L_HOPS:
                    c = tab_ref[50 + s]
                    rs_l[s, :, :] = (
                        rs_l[s, :, :].astype(jnp.float32)
                        + ybuf[pl.ds(c * B, B), :]).astype(jnp.bfloat16)
                    if len(rsl) >= 2:
                        rsl[-2].wait_send()
                    rsl.append(send_to(rs_l.at[s], rs_l.at[s + 1],
                                       rs_sl.at[len(rsl) % 2],
                                       rs_rl.at[s + 1], left))
            drain(rsr)
            drain(rsl)
            x_bs = (rs_r[R_HOPS - 1, :, :].astype(jnp.float32)
                    + rs_l[L_HOPS - 1, :, :].astype(jnp.float32)
                    + ybuf[pl.ds(i * B, B), :])

        out_ref[...] = x_bs

    return pl.pallas_call(
        body,
        out_shape=jax.ShapeDtypeStruct((B, D), jnp.float32),
        in_specs=[pl.BlockSpec(memory_space=pltpu.SMEM)]
        + [pl.BlockSpec(memory_space=pltpu.VMEM)] * 7,
        out_specs=pl.BlockSpec(memory_space=pltpu.VMEM),
        scratch_shapes=[
            pltpu.VMEM((M, D), jnp.bfloat16),
            pltpu.VMEM((M, D), jnp.float32),
            pltpu.VMEM((R_HOPS, B, D), jnp.bfloat16),
            pltpu.VMEM((L_HOPS, B, D), jnp.bfloat16),
            pltpu.VMEM((B, D), jnp.bfloat16),
            pltpu.VMEM((B, D), jnp.bfloat16),
            pltpu.SemaphoreType.DMA((2,)),
            pltpu.SemaphoreType.DMA((2,)),
            pltpu.SemaphoreType.DMA((2,)),
            pltpu.SemaphoreType.DMA((2,)),
            pltpu.SemaphoreType.DMA((N_DEV,)),
            pltpu.SemaphoreType.DMA((R_HOPS,)),
            pltpu.SemaphoreType.DMA((L_HOPS,)),
        ],
        compiler_params=pltpu.CompilerParams(collective_id=0),
    )(tab, x, Win0, Wout0, Win1, Wout1, Win2, Wout2)


# device time: 282741 ns/iter; 3.1821x vs baseline; 1.1028x over previous
import jax
import jax.numpy as jnp
from jax import lax
from jax.experimental import pallas as pl
from jax.experimental.pallas import tpu as pltpu

N_DEV = 32
R_HOPS = 16
L_HOPS = 15


def _r2i_py(r):
    r %= N_DEV
    q, zz = r // 4, r % 4
    z = zz if q % 2 == 0 else 3 - zz
    x = 0 if q < 4 else 1
    y = q if q < 4 else 7 - q
    xx = x if y % 2 == 0 else 1 - x
    return z * 8 + y * 2 + xx


def _i2r_py(i):
    z, y, xx = i // 8, (i % 8) // 2, i % 2
    x = xx if y % 2 == 0 else 1 - xx
    q = y if x == 0 else 7 - y
    return q * 4 + (z if q % 2 == 0 else 3 - z)


def kernel(x, Win0, Wout0, Win1, Wout1, Win2, Wout2):
    B, D = x.shape
    M = N_DEV * B

    r2i_tab = jnp.array([_r2i_py(r) for r in range(2 * N_DEV)],
                        dtype=jnp.int32)
    i2r_tab = jnp.array([_i2r_py(v) for v in range(N_DEV)],
                        dtype=jnp.int32)
    ii = lax.axis_index("i")
    r = i2r_tab[ii]

    def ring(e):
        return r2i_tab[(e + 2 * N_DEV) % (2 * N_DEV)]

    tab = jnp.concatenate([
        jnp.stack([ring(r - 1),
                   ring(r + 1),
                   ring(r + R_HOPS),
                   ring(r - L_HOPS)]),
        ring(r - 1 - jnp.arange(R_HOPS)),
        ring(r + 1 + jnp.arange(L_HOPS)),
        ring(r + R_HOPS - 1 - jnp.arange(L_HOPS)),
        ring(r - L_HOPS + 1 + jnp.arange(L_HOPS - 1)),
    ]).astype(jnp.int32)

    def body(tab_ref, x_ref, win0, wout0, win1, wout1, win2, wout2,
             out_ref, xfull, ybuf, rs_r, rs_l, stage_r, stage_l,
             ag_sr, ag_sl, rs_sr, rs_sl, ag_recv0, ag_recv1,
             rs_rr, rs_rl):
        i = lax.axis_index("i")
        left = tab_ref[0]
        right = tab_ref[1]

        barrier = pltpu.get_barrier_semaphore()
        for nbr in (left, right):
            pl.semaphore_signal(barrier, inc=1, device_id=(nbr,),
                                device_id_type=pl.DeviceIdType.MESH)
        pl.semaphore_wait(barrier, 2)

        def mod(e):
            return lax.rem(e + 2 * N_DEV, N_DEV)

        def chunk(ref, c):
            return ref.at[pl.ds(c * B, B), :]

        def send_to(src, dst, ssem, rsem, dev):
            r = pltpu.make_async_remote_copy(
                src_ref=src, dst_ref=dst, send_sem=ssem, recv_sem=rsem,
                device_id=(dev,), device_id_type=pl.DeviceIdType.MESH)
            r.start()
            return r

        def wait_recv(dst, rsem, dev):
            pltpu.make_async_remote_copy(
                src_ref=dst, dst_ref=dst, send_sem=rsem, recv_sem=rsem,
                device_id=(dev,), device_id_type=pl.DeviceIdType.MESH
            ).wait_recv()

        def drain(sends):
            for s in sends[-2:]:
                s.wait_send()

        x_bs = x_ref[...]
        for win, wout in ((win0, wout0), (win1, wout1), (win2, wout2)):

            def compute_chunk(c, win=win, wout=wout):
                xv = xfull[pl.ds(c * B, B), :].astype(jnp.float32)
                hv = jnp.maximum(
                    jnp.dot(xv, win[...],
                            preferred_element_type=jnp.float32), 0.0)
                ybuf[pl.ds(c * B, B), :] = jnp.dot(
                    hv, wout[...], preferred_element_type=jnp.float32)

            SB = B // 2

            def sub(c, k):
                return xfull.at[pl.ds(c * B + k * SB, SB), :]

            def agsem(k):
                return ag_recv0 if k == 0 else ag_recv1

            xfull[pl.ds(i * B, B), :] = x_bs.astype(jnp.bfloat16)
            sr, sl = [], []
            for k in range(2):
                sr.append(send_to(sub(i, k), sub(i, k),
                                  ag_sr.at[len(sr) % 2],
                                  agsem(k).at[i], right))
            for k in range(2):
                sl.append(send_to(sub(i, k), sub(i, k),
                                  ag_sl.at[len(sl) % 2],
                                  agsem(k).at[i], left))
            compute_chunk(i)
            for h in range(R_HOPS):
                c_l = tab_ref[4 + h]
                c_r = tab_ref[20 + h]
                for k in range(2):
                    wait_recv(sub(c_l, k), agsem(k).at[c_l], left)
                    if h + 1 < R_HOPS:
                        if len(sr) >= 2:
                            sr[-2].wait_send()
                        sr.append(send_to(sub(c_l, k), sub(c_l, k),
                                          ag_sr.at[len(sr) % 2],
                                          agsem(k).at[c_l], right))
                if h < L_HOPS:
                    for k in range(2):
                        wait_recv(sub(c_r, k), agsem(k).at[c_r], right)
                        if h + 1 < L_HOPS:
                            if len(sl) >= 2:
                                sl[-2].wait_send()
                            sl.append(send_to(sub(c_r, k), sub(c_r, k),
                                              ag_sl.at[len(sl) % 2],
                                              agsem(k).at[c_r], left))
                compute_chunk(c_l)
                if h < L_HOPS:
                    compute_chunk(c_r)
            drain(sr)
            drain(sl)

            stage_r[...] = ybuf[
                pl.ds(tab_ref[2] * B, B), :].astype(jnp.bfloat16)
            stage_l[...] = ybuf[
                pl.ds(tab_ref[3] * B, B), :].astype(jnp.bfloat16)
            rsr = [send_to(stage_r, rs_r.at[0],
                           rs_sr.at[0], rs_rr.at[0], right)]
            rsl = [send_to(stage_l, rs_l.at[0],
                           rs_sl.at[0], rs_rl.at[0], left)]
            for s in range(R_HOPS):
                wait_recv(rs_r.at[s], rs_rr.at[s], left)
                if s + 1 < R_HOPS:
                    c = tab_ref[35 + s]
                    rs_r[s, :, :] = (
                        rs_r[s, :, :].astype(jnp.float32)
                        + ybuf[pl.ds(c * B, B), :]).astype(jnp.bfloat16)
                    if len(rsr) >= 2:
                        rsr[-2].wait_send()
                    rsr.append(send_to(rs_r.at[s], rs_r.at[s + 1],
                                       rs_sr.at[len(rsr) % 2],
                                       rs_rr.at[s + 1], right))
                if s < L_HOPS:
                    wait_recv(rs_l.at[s], rs_rl.at[s], right)
                if s + 1 < L_HOPS:
                    c = tab_ref[50 + s]
                    rs_l[s, :, :] = (
                        rs_l[s, :, :].astype(jnp.float32)
                        + ybuf[pl.ds(c * B, B), :]).astype(jnp.bfloat16)
                    if len(rsl) >= 2:
                        rsl[-2].wait_send()
                    rsl.append(send_to(rs_l.at[s], rs_l.at[s + 1],
                                       rs_sl.at[len(rsl) % 2],
                                       rs_rl.at[s + 1], left))
            drain(rsr)
            drain(rsl)
            x_bs = (rs_r[R_HOPS - 1, :, :].astype(jnp.float32)
                    + rs_l[L_HOPS - 1, :, :].astype(jnp.float32)
                    + ybuf[pl.ds(i * B, B), :])

        out_ref[...] = x_bs

    return pl.pallas_call(
        body,
        out_shape=jax.ShapeDtypeStruct((B, D), jnp.float32),
        in_specs=[pl.BlockSpec(memory_space=pltpu.SMEM)]
        + [pl.BlockSpec(memory_space=pltpu.VMEM)] * 7,
        out_specs=pl.BlockSpec(memory_space=pltpu.VMEM),
        scratch_shapes=[
            pltpu.VMEM((M, D), jnp.bfloat16),
            pltpu.VMEM((M, D), jnp.float32),
            pltpu.VMEM((R_HOPS, B, D), jnp.bfloat16),
            pltpu.VMEM((L_HOPS, B, D), jnp.bfloat16),
            pltpu.VMEM((B, D), jnp.bfloat16),
            pltpu.VMEM((B, D), jnp.bfloat16),
            pltpu.SemaphoreType.DMA((2,)),
            pltpu.SemaphoreType.DMA((2,)),
            pltpu.SemaphoreType.DMA((2,)),
            pltpu.SemaphoreType.DMA((2,)),
            pltpu.SemaphoreType.DMA((N_DEV,)),
            pltpu.SemaphoreType.DMA((N_DEV,)),
            pltpu.SemaphoreType.DMA((R_HOPS,)),
            pltpu.SemaphoreType.DMA((L_HOPS,)),
        ],
        compiler_params=pltpu.CompilerParams(collective_id=0),
    )(tab, x, Win0, Wout0, Win1, Wout1, Win2, Wout2)
